# baseline (speedup 1.0000x reference)
import functools

import jax
import jax.numpy as jnp
from jax import lax
from jax.experimental import pallas as pl
from jax.experimental.pallas import tpu as pltpu
from jax.experimental.pallas import tpu_sc as plsc

N_NODES = 10000
D = 128

NC = 2
NS = 16
B = 128
NSTAGE = 2
CHUNKS = 40
NBUF = 2
E_PAD = NC * NS * NSTAGE * CHUNKS * B
ACC_ROWS = 10008
ZERO_ROWS_PER_TILE = 624
ZERO_ROWS_LAST = ACC_ROWS - 15 * ZERO_ROWS_PER_TILE
OUT_ROWS_PER_TILE = 624
OUT_ROWS_LAST = N_NODES - 15 * OUT_ROWS_PER_TILE


def _sc_segment_sum(src_r, dst_r, h, zeros_chunk):
  mesh = plsc.VectorSubcoreMesh(core_axis_name="c", subcore_axis_name="s")

  @functools.partial(
      pl.kernel,
      mesh=mesh,
      out_type=jax.ShapeDtypeStruct((NC, N_NODES, D), jnp.float32),
      scratch_types=[
          pltpu.VMEM((CHUNKS, B), jnp.int32),
          pltpu.VMEM((CHUNKS, B), jnp.int32),
          pltpu.VMEM((B, D), jnp.float32),
          pltpu.VMEM_SHARED((ACC_ROWS, D), jnp.float32),
          pltpu.SemaphoreType.DMA,
      ],
  )
  def seg_sum(src_hbm, dst_hbm, h_hbm, z_hbm, out_hbm,
              src_v, dst_v, rows_v, acc_sh, sem):
    c = lax.axis_index("c")
    s = lax.axis_index("s")

    @pl.when(s < NS - 1)
    def _():
      pltpu.sync_copy(
          z_hbm.at[pl.ds(0, ZERO_ROWS_PER_TILE)],
          acc_sh.at[pl.ds(s * ZERO_ROWS_PER_TILE, ZERO_ROWS_PER_TILE)])

    @pl.when(s == NS - 1)
    def _():
      pltpu.sync_copy(
          z_hbm,
          acc_sh.at[pl.ds((NS - 1) * ZERO_ROWS_PER_TILE, ZERO_ROWS_LAST)])

    plsc.subcore_barrier()

    for stage in range(NSTAGE):
      pltpu.sync_copy(src_hbm.at[c, s, stage], src_v)
      pltpu.sync_copy(dst_hbm.at[c, s, stage], dst_v)

      def body(j, carry):
        pltpu.async_copy(h_hbm.at[src_v.at[j]], rows_v, sem).wait()
        pltpu.sync_copy(rows_v, acc_sh.at[dst_v.at[j]], add=True)
        return carry

      lax.fori_loop(0, CHUNKS, body, 0)

    plsc.subcore_barrier()

    base = s * OUT_ROWS_PER_TILE

    @pl.when(s < NS - 1)
    def _():
      pltpu.sync_copy(acc_sh.at[pl.ds(base, OUT_ROWS_PER_TILE)],
                      out_hbm.at[c, pl.ds(base, OUT_ROWS_PER_TILE)])

    @pl.when(s == NS - 1)
    def _():
      last = (NS - 1) * OUT_ROWS_PER_TILE
      pltpu.sync_copy(acc_sh.at[pl.ds(last, OUT_ROWS_LAST)],
                      out_hbm.at[c, pl.ds(last, OUT_ROWS_LAST)])

  return seg_sum(src_r, dst_r, h, zeros_chunk)


def _tc_self_body(h_ref, ws_ref, b_ref, o_ref):
  acc = jnp.dot(h_ref[...], ws_ref[...], preferred_element_type=jnp.float32)
  o_ref[...] = acc + b_ref[...]


def _tc_self(h, w_self, b):
  blk = 1000
  return pl.pallas_call(
      _tc_self_body,
      grid=(N_NODES // blk,),
      in_specs=[
          pl.BlockSpec((blk, D), lambda i: (i, 0)),
          pl.BlockSpec((D, D), lambda i: (0, 0)),
          pl.BlockSpec((1, D), lambda i: (0, 0)),
      ],
      out_specs=pl.BlockSpec((blk, D), lambda i: (i, 0)),
      out_shape=jax.ShapeDtypeStruct((N_NODES, D), jnp.float32),
  )(h, w_self, b.reshape(1, D))


def _tc_combine_body(t_ref, p_ref, wn_ref, o_ref):
  agg = p_ref[0] + p_ref[1]
  acc = jnp.dot(agg, wn_ref[...], preferred_element_type=jnp.float32)
  o_ref[...] = jnp.maximum(acc + t_ref[...], 0.0)


def _tc_combine(t_self, partials, w_neigh):
  blk = 1000
  return pl.pallas_call(
      _tc_combine_body,
      grid=(N_NODES // blk,),
      in_specs=[
          pl.BlockSpec((blk, D), lambda i: (i, 0)),
          pl.BlockSpec((NC, blk, D), lambda i: (0, i, 0)),
          pl.BlockSpec((D, D), lambda i: (0, 0)),
      ],
      out_specs=pl.BlockSpec((blk, D), lambda i: (i, 0)),
      out_shape=jax.ShapeDtypeStruct((N_NODES, D), jnp.float32),
  )(t_self, partials, w_neigh)


def kernel(h, edge_index, W_self, W_neigh, b, index):
  del index
  src = edge_index[0].astype(jnp.int32)
  dst = edge_index[1].astype(jnp.int32)
  pad = E_PAD - src.shape[0]
  src_p = jnp.concatenate([src, jnp.zeros((pad,), jnp.int32)])
  dst_p = jnp.concatenate([dst, jnp.full((pad,), N_NODES, jnp.int32)])
  src_r = src_p.reshape(NC, NS, NSTAGE, CHUNKS, B)
  dst_r = dst_p.reshape(NC, NS, NSTAGE, CHUNKS, B)
  zeros_chunk = jnp.zeros((ZERO_ROWS_LAST, D), jnp.float32)
  partials = _sc_segment_sum(src_r, dst_r, h, zeros_chunk)
  t_self = _tc_self(h, W_self, b)
  return _tc_combine(t_self, partials, W_neigh)


# speedup vs baseline: 1.4389x; 1.4389x over previous
import functools

import jax
import jax.numpy as jnp
from jax import lax
from jax.experimental import pallas as pl
from jax.experimental.pallas import tpu as pltpu
from jax.experimental.pallas import tpu_sc as plsc

N_NODES = 10000
D = 128

NC = 2
NS = 16
B = 128
CHUNKS = 79
E_PAD = NC * NS * CHUNKS * B
ROWS_PER_TILE = 632
ACC_ROWS = NS * ROWS_PER_TILE
OUT_ROWS_PER_TILE = 624
OUT_ROWS_LAST = N_NODES - 15 * OUT_ROWS_PER_TILE


def _sc_segment_sum(src_r, dst_r, h, zeros_chunk):
  mesh = plsc.VectorSubcoreMesh(core_axis_name="c", subcore_axis_name="s")

  @functools.partial(
      pl.kernel,
      mesh=mesh,
      out_type=jax.ShapeDtypeStruct((NC, N_NODES, D), jnp.float32),
      scratch_types=[
          pltpu.VMEM((CHUNKS, B), jnp.int32),
          pltpu.VMEM((CHUNKS, B), jnp.int32),
          pltpu.VMEM((B, D), jnp.float32),
          pltpu.VMEM_SHARED((ACC_ROWS, D), jnp.float32),
          pltpu.SemaphoreType.DMA,
      ],
  )
  def seg_sum(src_hbm, dst_hbm, h_hbm, z_hbm, out_hbm,
              src_v, dst_v, rows_v, acc_sh, sem):
    c = lax.axis_index("c")
    s = lax.axis_index("s")

    pltpu.sync_copy(z_hbm, acc_sh.at[pl.ds(s * ROWS_PER_TILE, ROWS_PER_TILE)])
    pltpu.sync_copy(src_hbm.at[c, s], src_v)
    pltpu.sync_copy(dst_hbm.at[c, s], dst_v)
    plsc.subcore_barrier()

    def body(j, carry):
      pltpu.async_copy(h_hbm.at[src_v.at[j]], rows_v, sem).wait()
      pltpu.sync_copy(rows_v, acc_sh.at[dst_v.at[j]], add=True)
      return carry

    lax.fori_loop(0, CHUNKS, body, 0)
    plsc.subcore_barrier()

    base = s * OUT_ROWS_PER_TILE

    @pl.when(s < NS - 1)
    def _():
      pltpu.sync_copy(acc_sh.at[pl.ds(base, OUT_ROWS_PER_TILE)],
                      out_hbm.at[c, pl.ds(base, OUT_ROWS_PER_TILE)])

    @pl.when(s == NS - 1)
    def _():
      last = (NS - 1) * OUT_ROWS_PER_TILE
      pltpu.sync_copy(acc_sh.at[pl.ds(last, OUT_ROWS_LAST)],
                      out_hbm.at[c, pl.ds(last, OUT_ROWS_LAST)])

  return seg_sum(src_r, dst_r, h, zeros_chunk)


def _tc_combine_body(h_ref, p_ref, ws_ref, wn_ref, b_ref, o_ref):
  agg = p_ref[0] + p_ref[1]
  acc = jnp.dot(h_ref[...], ws_ref[...], preferred_element_type=jnp.float32)
  acc = acc + jnp.dot(agg, wn_ref[...], preferred_element_type=jnp.float32)
  o_ref[...] = jnp.maximum(acc + b_ref[...], 0.0)


def _tc_combine(h, partials, w_self, w_neigh, b):
  blk = 1000
  grid = (N_NODES // blk,)
  return pl.pallas_call(
      _tc_combine_body,
      grid=grid,
      in_specs=[
          pl.BlockSpec((blk, D), lambda i: (i, 0)),
          pl.BlockSpec((NC, blk, D), lambda i: (0, i, 0)),
          pl.BlockSpec((D, D), lambda i: (0, 0)),
          pl.BlockSpec((D, D), lambda i: (0, 0)),
          pl.BlockSpec((1, D), lambda i: (0, 0)),
      ],
      out_specs=pl.BlockSpec((blk, D), lambda i: (i, 0)),
      out_shape=jax.ShapeDtypeStruct((N_NODES, D), jnp.float32),
  )(h, partials, w_self, w_neigh, b.reshape(1, D))


def kernel(h, edge_index, W_self, W_neigh, b, index):
  del index
  src = edge_index[0].astype(jnp.int32)
  dst = edge_index[1].astype(jnp.int32)
  pad = E_PAD - src.shape[0]
  src_p = jnp.concatenate([src, jnp.zeros((pad,), jnp.int32)])
  dst_p = jnp.concatenate([dst, jnp.full((pad,), N_NODES, jnp.int32)])
  src_r = src_p.reshape(NC, NS, CHUNKS, B)
  dst_r = dst_p.reshape(NC, NS, CHUNKS, B)
  zeros_chunk = jnp.zeros((ROWS_PER_TILE, D), jnp.float32)
  partials = _sc_segment_sum(src_r, dst_r, h, zeros_chunk)
  return _tc_combine(h, partials, W_self, W_neigh, b)


# speedup vs baseline: 1.4394x; 1.0003x over previous
import functools

import jax
import jax.numpy as jnp
from jax import lax
from jax.experimental import pallas as pl
from jax.experimental.pallas import tpu as pltpu
from jax.experimental.pallas import tpu_sc as plsc

N_NODES = 10000
D = 128

NC = 2
NS = 16
B = 128
CHUNKS = 79
E_PAD = NC * NS * CHUNKS * B
ROWS_PER_TILE = 632
ACC_ROWS = NS * ROWS_PER_TILE
OUT_ROWS_PER_TILE = 624
OUT_ROWS_LAST = N_NODES - 15 * OUT_ROWS_PER_TILE


def _sc_segment_sum(src_r, dst_r, h, zeros_chunk):
  mesh = plsc.VectorSubcoreMesh(core_axis_name="c", subcore_axis_name="s")

  @functools.partial(
      pl.kernel,
      mesh=mesh,
      out_type=jax.ShapeDtypeStruct((NC, N_NODES, D), jnp.float32),
      scratch_types=[
          pltpu.VMEM((CHUNKS, B), jnp.int32),
          pltpu.VMEM((CHUNKS, B), jnp.int32),
          pltpu.VMEM((B, D), jnp.float32),
          pltpu.VMEM_SHARED((ACC_ROWS, D), jnp.float32),
          pltpu.SemaphoreType.DMA,
      ],
  )
  def seg_sum(src_hbm, dst_hbm, h_hbm, z_hbm, out_hbm,
              src_v, dst_v, rows_v, acc_sh, sem):
    c = lax.axis_index("c")
    s = lax.axis_index("s")

    pltpu.sync_copy(z_hbm, acc_sh.at[pl.ds(s * ROWS_PER_TILE, ROWS_PER_TILE)])
    pltpu.sync_copy(src_hbm.at[c, s], src_v)
    pltpu.sync_copy(dst_hbm.at[c, s], dst_v)
    plsc.subcore_barrier()

    def body(j, carry):
      pltpu.async_copy(h_hbm.at[src_v.at[j]], rows_v, sem).wait()
      pltpu.sync_copy(rows_v, acc_sh.at[dst_v.at[j]], add=True)
      return carry

    lax.fori_loop(0, CHUNKS, body, 0)
    plsc.subcore_barrier()

    base = s * OUT_ROWS_PER_TILE

    @pl.when(s < NS - 1)
    def _():
      pltpu.sync_copy(acc_sh.at[pl.ds(base, OUT_ROWS_PER_TILE)],
                      out_hbm.at[c, pl.ds(base, OUT_ROWS_PER_TILE)])

    @pl.when(s == NS - 1)
    def _():
      last = (NS - 1) * OUT_ROWS_PER_TILE
      pltpu.sync_copy(acc_sh.at[pl.ds(last, OUT_ROWS_LAST)],
                      out_hbm.at[c, pl.ds(last, OUT_ROWS_LAST)])

  return seg_sum(src_r, dst_r, h, zeros_chunk)


def _tc_combine_body(h_ref, p_ref, ws_ref, wn_ref, b_ref, o_ref):
  agg = p_ref[0] + p_ref[1]
  acc = jnp.dot(h_ref[...], ws_ref[...], preferred_element_type=jnp.float32)
  acc = acc + jnp.dot(agg, wn_ref[...], preferred_element_type=jnp.float32)
  o_ref[...] = jnp.maximum(acc + b_ref[...], 0.0)


def _tc_combine(h, partials, w_self, w_neigh, b):
  blk = 1000
  grid = (N_NODES // blk,)
  return pl.pallas_call(
      _tc_combine_body,
      grid=grid,
      in_specs=[
          pl.BlockSpec((blk, D), lambda i: (i, 0)),
          pl.BlockSpec((NC, blk, D), lambda i: (0, i, 0)),
          pl.BlockSpec((D, D), lambda i: (0, 0)),
          pl.BlockSpec((D, D), lambda i: (0, 0)),
          pl.BlockSpec((1, D), lambda i: (0, 0)),
      ],
      out_specs=pl.BlockSpec((blk, D), lambda i: (i, 0)),
      out_shape=jax.ShapeDtypeStruct((N_NODES, D), jnp.float32),
  )(h, partials, w_self, w_neigh, b.reshape(1, D))


def kernel(h, edge_index, W_self, W_neigh, b, index):
  del index
  src = edge_index[0].astype(jnp.int32)
  dst = edge_index[1].astype(jnp.int32)
  pad = E_PAD - src.shape[0]
  spare = ACC_ROWS - N_NODES
  pad_dst = N_NODES + (jnp.arange(pad, dtype=jnp.int32) % spare)
  src_p = jnp.concatenate([src, jnp.zeros((pad,), jnp.int32)])
  dst_p = jnp.concatenate([dst, pad_dst])
  src_r = src_p.reshape(NC, NS, CHUNKS, B)
  dst_r = dst_p.reshape(NC, NS, CHUNKS, B)
  zeros_chunk = jnp.zeros((ROWS_PER_TILE, D), jnp.float32)
  partials = _sc_segment_sum(src_r, dst_r, h, zeros_chunk)
  return _tc_combine(h, partials, W_self, W_neigh, b)


# speedup vs baseline: 1.7693x; 1.2292x over previous
import functools

import jax
import jax.numpy as jnp
from jax import lax
from jax.experimental import pallas as pl
from jax.experimental.pallas import tpu as pltpu
from jax.experimental.pallas import tpu_sc as plsc

N_NODES = 10000
D = 128

NC = 2
NS = 16
B = 128
CHUNKS0 = 97
CHUNKS1 = 60
E0 = NS * CHUNKS0 * B
E1 = NS * CHUNKS1 * B
E_PAD = E0 + E1
ROWS_PER_TILE = 632
ACC_ROWS = NS * ROWS_PER_TILE
OUT_ROWS_PER_TILE = 624
OUT_ROWS_LAST = N_NODES - 15 * OUT_ROWS_PER_TILE


def _sc_segment_sum(src_r, dst_r, h, zeros_chunk):
  mesh = plsc.VectorSubcoreMesh(core_axis_name="c", subcore_axis_name="s")

  @functools.partial(
      pl.kernel,
      mesh=mesh,
      out_type=jax.ShapeDtypeStruct((NC, N_NODES, D), jnp.float32),
      scratch_types=[
          pltpu.VMEM((CHUNKS0, B), jnp.int32),
          pltpu.VMEM((CHUNKS0, B), jnp.int32),
          pltpu.VMEM((B, D), jnp.float32),
          pltpu.VMEM_SHARED((ACC_ROWS, D), jnp.float32),
          pltpu.SemaphoreType.DMA,
      ],
  )
  def seg_sum(src_hbm, dst_hbm, h_hbm, z_hbm, out_hbm,
              src_v, dst_v, rows_v, acc_sh, sem):
    c = lax.axis_index("c")
    s = lax.axis_index("s")

    pltpu.sync_copy(z_hbm, acc_sh.at[pl.ds(s * ROWS_PER_TILE, ROWS_PER_TILE)])
    pltpu.sync_copy(src_hbm.at[c, s], src_v)
    pltpu.sync_copy(dst_hbm.at[c, s], dst_v)
    plsc.subcore_barrier()

    def body(j, carry):
      pltpu.async_copy(h_hbm.at[src_v.at[j]], rows_v, sem).wait()
      pltpu.sync_copy(rows_v, acc_sh.at[dst_v.at[j]], add=True)
      return carry

    nchunks = jnp.where(c == 0, CHUNKS0, CHUNKS1)
    lax.fori_loop(0, nchunks, body, 0)
    plsc.subcore_barrier()

    base = s * OUT_ROWS_PER_TILE

    @pl.when(s < NS - 1)
    def _():
      pltpu.sync_copy(acc_sh.at[pl.ds(base, OUT_ROWS_PER_TILE)],
                      out_hbm.at[c, pl.ds(base, OUT_ROWS_PER_TILE)])

    @pl.when(s == NS - 1)
    def _():
      last = (NS - 1) * OUT_ROWS_PER_TILE
      pltpu.sync_copy(acc_sh.at[pl.ds(last, OUT_ROWS_LAST)],
                      out_hbm.at[c, pl.ds(last, OUT_ROWS_LAST)])

  return seg_sum(src_r, dst_r, h, zeros_chunk)


def _tc_combine_body(h_ref, p_ref, ws_ref, wn_ref, b_ref, o_ref):
  agg = p_ref[0] + p_ref[1]
  acc = jnp.dot(h_ref[...], ws_ref[...], preferred_element_type=jnp.float32)
  acc = acc + jnp.dot(agg, wn_ref[...], preferred_element_type=jnp.float32)
  o_ref[...] = jnp.maximum(acc + b_ref[...], 0.0)


def _tc_combine(h, partials, w_self, w_neigh, b):
  blk = 1000
  grid = (N_NODES // blk,)
  return pl.pallas_call(
      _tc_combine_body,
      grid=grid,
      in_specs=[
          pl.BlockSpec((blk, D), lambda i: (i, 0)),
          pl.BlockSpec((NC, blk, D), lambda i: (0, i, 0)),
          pl.BlockSpec((D, D), lambda i: (0, 0)),
          pl.BlockSpec((D, D), lambda i: (0, 0)),
          pl.BlockSpec((1, D), lambda i: (0, 0)),
      ],
      out_specs=pl.BlockSpec((blk, D), lambda i: (i, 0)),
      out_shape=jax.ShapeDtypeStruct((N_NODES, D), jnp.float32),
  )(h, partials, w_self, w_neigh, b.reshape(1, D))


def kernel(h, edge_index, W_self, W_neigh, b, index):
  del index
  src = edge_index[0].astype(jnp.int32)
  dst = edge_index[1].astype(jnp.int32)
  pad = E_PAD - src.shape[0]
  spare = ACC_ROWS - N_NODES
  pad_dst = N_NODES + (jnp.arange(pad, dtype=jnp.int32) % spare)
  src_p = jnp.concatenate([src, jnp.zeros((pad,), jnp.int32)])
  dst_p = jnp.concatenate([dst, pad_dst])

  def split(x):
    x0 = x[:E0].reshape(NS, CHUNKS0, B)
    x1 = x[E0:].reshape(NS, CHUNKS1, B)
    x1 = jnp.pad(x1, ((0, 0), (0, CHUNKS0 - CHUNKS1), (0, 0)))
    return jnp.stack([x0, x1])

  src_r = split(src_p)
  dst_r = split(dst_p)
  zeros_chunk = jnp.zeros((ROWS_PER_TILE, D), jnp.float32)
  partials = _sc_segment_sum(src_r, dst_r, h, zeros_chunk)
  return _tc_combine(h, partials, W_self, W_neigh, b)


# speedup vs baseline: 1.7956x; 1.0149x over previous
import functools

import jax
import jax.numpy as jnp
from jax import lax
from jax.experimental import pallas as pl
from jax.experimental.pallas import tpu as pltpu
from jax.experimental.pallas import tpu_sc as plsc

N_NODES = 10000
D = 128

NC = 2
NS = 16
B = 128
CHUNKS0 = 95
CHUNKS1 = 62
E0 = NS * CHUNKS0 * B
E1 = NS * CHUNKS1 * B
E_PAD = E0 + E1
ROWS_PER_TILE = 632
ACC_ROWS = NS * ROWS_PER_TILE
OUT_ROWS_PER_TILE = 624
OUT_ROWS_LAST = N_NODES - 15 * OUT_ROWS_PER_TILE


def _sc_segment_sum(src_r, dst_r, h, zeros_chunk):
  mesh = plsc.VectorSubcoreMesh(core_axis_name="c", subcore_axis_name="s")

  @functools.partial(
      pl.kernel,
      mesh=mesh,
      out_type=jax.ShapeDtypeStruct((NC, N_NODES, D), jnp.float32),
      scratch_types=[
          pltpu.VMEM((CHUNKS0, B), jnp.int32),
          pltpu.VMEM((CHUNKS0, B), jnp.int32),
          pltpu.VMEM((B, D), jnp.float32),
          pltpu.VMEM_SHARED((ACC_ROWS, D), jnp.float32),
          pltpu.SemaphoreType.DMA,
      ],
  )
  def seg_sum(src_hbm, dst_hbm, h_hbm, z_hbm, out_hbm,
              src_v, dst_v, rows_v, acc_sh, sem):
    c = lax.axis_index("c")
    s = lax.axis_index("s")

    pltpu.sync_copy(z_hbm, acc_sh.at[pl.ds(s * ROWS_PER_TILE, ROWS_PER_TILE)])
    pltpu.sync_copy(src_hbm.at[c, s], src_v)
    pltpu.sync_copy(dst_hbm.at[c, s], dst_v)
    plsc.subcore_barrier()

    def body(j, carry):
      pltpu.async_copy(h_hbm.at[src_v.at[j]], rows_v, sem).wait()
      pltpu.sync_copy(rows_v, acc_sh.at[dst_v.at[j]], add=True)
      return carry

    nchunks = jnp.where(c == 0, CHUNKS0, CHUNKS1)
    lax.fori_loop(0, nchunks, body, 0)
    plsc.subcore_barrier()

    base = s * OUT_ROWS_PER_TILE

    @pl.when(s < NS - 1)
    def _():
      pltpu.sync_copy(acc_sh.at[pl.ds(base, OUT_ROWS_PER_TILE)],
                      out_hbm.at[c, pl.ds(base, OUT_ROWS_PER_TILE)])

    @pl.when(s == NS - 1)
    def _():
      last = (NS - 1) * OUT_ROWS_PER_TILE
      pltpu.sync_copy(acc_sh.at[pl.ds(last, OUT_ROWS_LAST)],
                      out_hbm.at[c, pl.ds(last, OUT_ROWS_LAST)])

  return seg_sum(src_r, dst_r, h, zeros_chunk)


def _tc_combine_body(h_ref, p_ref, ws_ref, wn_ref, b_ref, o_ref):
  agg = p_ref[0] + p_ref[1]
  acc = jnp.dot(h_ref[...], ws_ref[...], preferred_element_type=jnp.float32)
  acc = acc + jnp.dot(agg, wn_ref[...], preferred_element_type=jnp.float32)
  o_ref[...] = jnp.maximum(acc + b_ref[...], 0.0)


def _tc_combine(h, partials, w_self, w_neigh, b):
  blk = 1000
  grid = (N_NODES // blk,)
  return pl.pallas_call(
      _tc_combine_body,
      grid=grid,
      in_specs=[
          pl.BlockSpec((blk, D), lambda i: (i, 0)),
          pl.BlockSpec((NC, blk, D), lambda i: (0, i, 0)),
          pl.BlockSpec((D, D), lambda i: (0, 0)),
          pl.BlockSpec((D, D), lambda i: (0, 0)),
          pl.BlockSpec((1, D), lambda i: (0, 0)),
      ],
      out_specs=pl.BlockSpec((blk, D), lambda i: (i, 0)),
      out_shape=jax.ShapeDtypeStruct((N_NODES, D), jnp.float32),
  )(h, partials, w_self, w_neigh, b.reshape(1, D))


def kernel(h, edge_index, W_self, W_neigh, b, index):
  del index
  src = edge_index[0].astype(jnp.int32)
  dst = edge_index[1].astype(jnp.int32)
  pad = E_PAD - src.shape[0]
  spare = ACC_ROWS - N_NODES
  pad_dst = N_NODES + (jnp.arange(pad, dtype=jnp.int32) % spare)
  src_p = jnp.concatenate([src, jnp.zeros((pad,), jnp.int32)])
  dst_p = jnp.concatenate([dst, pad_dst])

  def split(x):
    x0 = x[:E0].reshape(NS, CHUNKS0, B)
    x1 = x[E0:].reshape(NS, CHUNKS1, B)
    x1 = jnp.pad(x1, ((0, 0), (0, CHUNKS0 - CHUNKS1), (0, 0)))
    return jnp.stack([x0, x1])

  src_r = split(src_p)
  dst_r = split(dst_p)
  zeros_chunk = jnp.zeros((ROWS_PER_TILE, D), jnp.float32)
  partials = _sc_segment_sum(src_r, dst_r, h, zeros_chunk)
  return _tc_combine(h, partials, W_self, W_neigh, b)


# speedup vs baseline: 1.8241x; 1.0159x over previous
import functools

import jax
import jax.numpy as jnp
from jax import lax
from jax.experimental import pallas as pl
from jax.experimental.pallas import tpu as pltpu
from jax.experimental.pallas import tpu_sc as plsc

N_NODES = 10000
D = 128

NC = 2
NS = 16
B = 128
CHUNKS0 = 95
CHUNKS1 = 62
E0 = NS * CHUNKS0 * B
E1 = NS * CHUNKS1 * B
E_PAD = E0 + E1
ROWS_PER_TILE = 632
ACC_ROWS = NS * ROWS_PER_TILE
OUT_ROWS_PER_TILE = 624
OUT_ROWS_LAST = N_NODES - 15 * OUT_ROWS_PER_TILE


def _sc_segment_sum(src_r, dst_r, h, zeros_chunk):
  mesh = plsc.VectorSubcoreMesh(core_axis_name="c", subcore_axis_name="s")

  @functools.partial(
      pl.kernel,
      mesh=mesh,
      out_type=jax.ShapeDtypeStruct((NC, N_NODES, D), jnp.float32),
      scratch_types=[
          pltpu.VMEM((CHUNKS0, B), jnp.int32),
          pltpu.VMEM((CHUNKS0, B), jnp.int32),
          pltpu.VMEM((B, D), jnp.float32),
          pltpu.VMEM_SHARED((ACC_ROWS, D), jnp.float32),
          pltpu.SemaphoreType.DMA,
      ],
  )
  def seg_sum(src_hbm, dst_hbm, h_hbm, z_hbm, out_hbm,
              src_v, dst_v, rows_v, acc_sh, sem):
    c = lax.axis_index("c")
    s = lax.axis_index("s")

    pltpu.sync_copy(z_hbm, acc_sh.at[pl.ds(s * ROWS_PER_TILE, ROWS_PER_TILE)])
    pltpu.sync_copy(src_hbm.at[c, s], src_v)
    pltpu.sync_copy(dst_hbm.at[c, s], dst_v)
    plsc.subcore_barrier()

    def body(j, carry):
      pltpu.async_copy(h_hbm.at[src_v.at[j]], rows_v, sem).wait()
      pltpu.sync_copy(rows_v, acc_sh.at[dst_v.at[j]], add=True)
      return carry

    nchunks = jnp.where(c == 0, CHUNKS0, CHUNKS1)
    lax.fori_loop(0, nchunks, body, 0)
    plsc.subcore_barrier()

    base = s * OUT_ROWS_PER_TILE

    @pl.when(s < NS - 1)
    def _():
      pltpu.sync_copy(acc_sh.at[pl.ds(base, OUT_ROWS_PER_TILE)],
                      out_hbm.at[c, pl.ds(base, OUT_ROWS_PER_TILE)])

    @pl.when(s == NS - 1)
    def _():
      last = (NS - 1) * OUT_ROWS_PER_TILE
      pltpu.sync_copy(acc_sh.at[pl.ds(last, OUT_ROWS_LAST)],
                      out_hbm.at[c, pl.ds(last, OUT_ROWS_LAST)])

  return seg_sum(src_r, dst_r, h, zeros_chunk)


def _tc_combine_body(h_ref, p_ref, ws_ref, wn_ref, b_ref, o_ref):
  agg = p_ref[0] + p_ref[1]
  acc = jnp.dot(h_ref[...], ws_ref[...], preferred_element_type=jnp.float32)
  acc = acc + jnp.dot(agg, wn_ref[...], preferred_element_type=jnp.float32)
  o_ref[...] = jnp.maximum(acc + b_ref[...], 0.0)


def _tc_combine(h, partials, w_self, w_neigh, b):
  blk = 2000
  grid = (N_NODES // blk,)
  return pl.pallas_call(
      _tc_combine_body,
      grid=grid,
      in_specs=[
          pl.BlockSpec((blk, D), lambda i: (i, 0)),
          pl.BlockSpec((NC, blk, D), lambda i: (0, i, 0)),
          pl.BlockSpec((D, D), lambda i: (0, 0)),
          pl.BlockSpec((D, D), lambda i: (0, 0)),
          pl.BlockSpec((1, D), lambda i: (0, 0)),
      ],
      out_specs=pl.BlockSpec((blk, D), lambda i: (i, 0)),
      out_shape=jax.ShapeDtypeStruct((N_NODES, D), jnp.float32),
  )(h, partials, w_self, w_neigh, b.reshape(1, D))


def kernel(h, edge_index, W_self, W_neigh, b, index):
  del index
  src = edge_index[0].astype(jnp.int32)
  dst = edge_index[1].astype(jnp.int32)
  pad = E_PAD - src.shape[0]
  spare = ACC_ROWS - N_NODES
  pad_dst = N_NODES + (jnp.arange(pad, dtype=jnp.int32) % spare)
  src_p = jnp.concatenate([src, jnp.zeros((pad,), jnp.int32)])
  dst_p = jnp.concatenate([dst, pad_dst])

  def split(x):
    x0 = x[:E0].reshape(NS, CHUNKS0, B)
    x1 = x[E0:].reshape(NS, CHUNKS1, B)
    x1 = jnp.pad(x1, ((0, 0), (0, CHUNKS0 - CHUNKS1), (0, 0)))
    return jnp.stack([x0, x1])

  src_r = split(src_p)
  dst_r = split(dst_p)
  zeros_chunk = jnp.zeros((ROWS_PER_TILE, D), jnp.float32)
  partials = _sc_segment_sum(src_r, dst_r, h, zeros_chunk)
  return _tc_combine(h, partials, W_self, W_neigh, b)


# speedup vs baseline: 2.1039x; 1.1534x over previous
import functools

import jax
import jax.numpy as jnp
from jax import lax
from jax.experimental import pallas as pl
from jax.experimental.pallas import tpu as pltpu
from jax.experimental.pallas import tpu_sc as plsc

N_NODES = 10000
N_EDGES = 320000
D = 128

NC = 2
NS = 16
B = 128
CHUNKS0 = 95
CHUNKS1 = 62
E0 = NS * CHUNKS0 * B
CHUNKS1_LAST = (N_EDGES - E0) // B - (NS - 1) * CHUNKS1
ROWS_PER_TILE = 632
ACC_ROWS = NS * ROWS_PER_TILE
OUT_ROWS_PER_TILE = 624
OUT_ROWS_LAST = N_NODES - 15 * OUT_ROWS_PER_TILE


def _sc_segment_sum(edges, h, zeros_chunk):
  mesh = plsc.VectorSubcoreMesh(core_axis_name="c", subcore_axis_name="s")

  @functools.partial(
      pl.kernel,
      mesh=mesh,
      out_type=jax.ShapeDtypeStruct((NC, N_NODES, D), jnp.float32),
      scratch_types=[
          pltpu.VMEM((CHUNKS0 * B,), jnp.int32),
          pltpu.VMEM((CHUNKS0 * B,), jnp.int32),
          pltpu.VMEM((B, D), jnp.float32),
          pltpu.VMEM_SHARED((ACC_ROWS, D), jnp.float32),
          pltpu.SemaphoreType.DMA,
      ],
  )
  def seg_sum(edges_hbm, h_hbm, z_hbm, out_hbm,
              src_v, dst_v, rows_v, acc_sh, sem):
    c = lax.axis_index("c")
    s = lax.axis_index("s")

    pltpu.sync_copy(z_hbm, acc_sh.at[pl.ds(s * ROWS_PER_TILE, ROWS_PER_TILE)])

    @pl.when(c == 0)
    def _():
      start = s * (CHUNKS0 * B)
      pltpu.sync_copy(edges_hbm.at[0, pl.ds(start, CHUNKS0 * B)],
                      src_v.at[pl.ds(0, CHUNKS0 * B)])
      pltpu.sync_copy(edges_hbm.at[1, pl.ds(start, CHUNKS0 * B)],
                      dst_v.at[pl.ds(0, CHUNKS0 * B)])

    @pl.when((c == 1) & (s < NS - 1))
    def _():
      start = E0 + s * (CHUNKS1 * B)
      pltpu.sync_copy(edges_hbm.at[0, pl.ds(start, CHUNKS1 * B)],
                      src_v.at[pl.ds(0, CHUNKS1 * B)])
      pltpu.sync_copy(edges_hbm.at[1, pl.ds(start, CHUNKS1 * B)],
                      dst_v.at[pl.ds(0, CHUNKS1 * B)])

    @pl.when((c == 1) & (s == NS - 1))
    def _():
      start = E0 + (NS - 1) * (CHUNKS1 * B)
      pltpu.sync_copy(edges_hbm.at[0, pl.ds(start, CHUNKS1_LAST * B)],
                      src_v.at[pl.ds(0, CHUNKS1_LAST * B)])
      pltpu.sync_copy(edges_hbm.at[1, pl.ds(start, CHUNKS1_LAST * B)],
                      dst_v.at[pl.ds(0, CHUNKS1_LAST * B)])

    plsc.subcore_barrier()

    def body(j, carry):
      pltpu.async_copy(h_hbm.at[src_v.at[pl.ds(j * B, B)]], rows_v, sem).wait()
      pltpu.sync_copy(rows_v, acc_sh.at[dst_v.at[pl.ds(j * B, B)]], add=True)
      return carry

    nchunks = jnp.where(c == 0, CHUNKS0,
                        jnp.where(s == NS - 1, CHUNKS1_LAST, CHUNKS1))
    lax.fori_loop(0, nchunks, body, 0)
    plsc.subcore_barrier()

    base = s * OUT_ROWS_PER_TILE

    @pl.when(s < NS - 1)
    def _():
      pltpu.sync_copy(acc_sh.at[pl.ds(base, OUT_ROWS_PER_TILE)],
                      out_hbm.at[c, pl.ds(base, OUT_ROWS_PER_TILE)])

    @pl.when(s == NS - 1)
    def _():
      last = (NS - 1) * OUT_ROWS_PER_TILE
      pltpu.sync_copy(acc_sh.at[pl.ds(last, OUT_ROWS_LAST)],
                      out_hbm.at[c, pl.ds(last, OUT_ROWS_LAST)])

  return seg_sum(edges, h, zeros_chunk)


def _tc_combine_body(h_ref, p_ref, ws_ref, wn_ref, b_ref, o_ref):
  agg = p_ref[0] + p_ref[1]
  acc = jnp.dot(h_ref[...], ws_ref[...], preferred_element_type=jnp.float32)
  acc = acc + jnp.dot(agg, wn_ref[...], preferred_element_type=jnp.float32)
  o_ref[...] = jnp.maximum(acc + b_ref[...], 0.0)


def _tc_combine(h, partials, w_self, w_neigh, b):
  blk = 2000
  grid = (N_NODES // blk,)
  return pl.pallas_call(
      _tc_combine_body,
      grid=grid,
      in_specs=[
          pl.BlockSpec((blk, D), lambda i: (i, 0)),
          pl.BlockSpec((NC, blk, D), lambda i: (0, i, 0)),
          pl.BlockSpec((D, D), lambda i: (0, 0)),
          pl.BlockSpec((D, D), lambda i: (0, 0)),
          pl.BlockSpec((1, D), lambda i: (0, 0)),
      ],
      out_specs=pl.BlockSpec((blk, D), lambda i: (i, 0)),
      out_shape=jax.ShapeDtypeStruct((N_NODES, D), jnp.float32),
  )(h, partials, w_self, w_neigh, b.reshape(1, D))


def kernel(h, edge_index, W_self, W_neigh, b, index):
  del index
  edges = edge_index.astype(jnp.int32)
  zeros_chunk = jnp.zeros((ROWS_PER_TILE, D), jnp.float32)
  partials = _sc_segment_sum(edges, h, zeros_chunk)
  return _tc_combine(h, partials, W_self, W_neigh, b)


# speedup vs baseline: 2.4207x; 1.1506x over previous
import functools

import jax
import jax.numpy as jnp
from jax import lax
from jax.experimental import pallas as pl
from jax.experimental.pallas import tpu as pltpu
from jax.experimental.pallas import tpu_sc as plsc

N_NODES = 10000
N_EDGES = 320000
D = 128

NC = 2
NS = 16
B = 128
CHUNKS0 = 79
CHUNKS1 = 78
E0 = NS * CHUNKS0 * B
CHUNKS1_LAST = (N_EDGES - E0) // B - (NS - 1) * CHUNKS1
ROWS_PER_TILE = 632
ACC_ROWS = NS * ROWS_PER_TILE
OUT_ROWS_PER_TILE = 624
OUT_ROWS_LAST = N_NODES - 15 * OUT_ROWS_PER_TILE


def _sc_segment_sum(edges, h, zeros_chunk):
  mesh = plsc.VectorSubcoreMesh(core_axis_name="c", subcore_axis_name="s")

  @functools.partial(
      pl.kernel,
      mesh=mesh,
      out_type=jax.ShapeDtypeStruct((NC, N_NODES, D), jnp.float32),
      scratch_types=[
          pltpu.VMEM((CHUNKS0 * B,), jnp.int32),
          pltpu.VMEM((CHUNKS0 * B,), jnp.int32),
          pltpu.VMEM((B, D), jnp.float32),
          pltpu.VMEM_SHARED((ACC_ROWS, D), jnp.float32),
          pltpu.SemaphoreType.DMA,
      ],
  )
  def seg_sum(edges_hbm, h_hbm, z_hbm, out_hbm,
              src_v, dst_v, rows_v, acc_sh, sem):
    c = lax.axis_index("c")
    s = lax.axis_index("s")

    pltpu.sync_copy(z_hbm, acc_sh.at[pl.ds(s * ROWS_PER_TILE, ROWS_PER_TILE)])

    @pl.when(c == 0)
    def _():
      start = s * (CHUNKS0 * B)
      pltpu.sync_copy(edges_hbm.at[0, pl.ds(start, CHUNKS0 * B)],
                      src_v.at[pl.ds(0, CHUNKS0 * B)])
      pltpu.sync_copy(edges_hbm.at[1, pl.ds(start, CHUNKS0 * B)],
                      dst_v.at[pl.ds(0, CHUNKS0 * B)])

    @pl.when((c == 1) & (s < NS - 1))
    def _():
      start = E0 + s * (CHUNKS1 * B)
      pltpu.sync_copy(edges_hbm.at[0, pl.ds(start, CHUNKS1 * B)],
                      src_v.at[pl.ds(0, CHUNKS1 * B)])
      pltpu.sync_copy(edges_hbm.at[1, pl.ds(start, CHUNKS1 * B)],
                      dst_v.at[pl.ds(0, CHUNKS1 * B)])

    @pl.when((c == 1) & (s == NS - 1))
    def _():
      start = E0 + (NS - 1) * (CHUNKS1 * B)
      pltpu.sync_copy(edges_hbm.at[0, pl.ds(start, CHUNKS1_LAST * B)],
                      src_v.at[pl.ds(0, CHUNKS1_LAST * B)])
      pltpu.sync_copy(edges_hbm.at[1, pl.ds(start, CHUNKS1_LAST * B)],
                      dst_v.at[pl.ds(0, CHUNKS1_LAST * B)])

    plsc.subcore_barrier()

    def body(j, carry):
      pltpu.async_copy(h_hbm.at[src_v.at[pl.ds(j * B, B)]], rows_v, sem).wait()
      pltpu.sync_copy(rows_v, acc_sh.at[dst_v.at[pl.ds(j * B, B)]], add=True)
      return carry

    nchunks = jnp.where(c == 0, CHUNKS0,
                        jnp.where(s == NS - 1, CHUNKS1_LAST, CHUNKS1))
    lax.fori_loop(0, nchunks, body, 0)
    plsc.subcore_barrier()

    base = s * OUT_ROWS_PER_TILE

    @pl.when(s < NS - 1)
    def _():
      pltpu.sync_copy(acc_sh.at[pl.ds(base, OUT_ROWS_PER_TILE)],
                      out_hbm.at[c, pl.ds(base, OUT_ROWS_PER_TILE)])

    @pl.when(s == NS - 1)
    def _():
      last = (NS - 1) * OUT_ROWS_PER_TILE
      pltpu.sync_copy(acc_sh.at[pl.ds(last, OUT_ROWS_LAST)],
                      out_hbm.at[c, pl.ds(last, OUT_ROWS_LAST)])

  return seg_sum(edges, h, zeros_chunk)


def _tc_combine_body(h_ref, p_ref, ws_ref, wn_ref, b_ref, o_ref):
  agg = p_ref[0] + p_ref[1]
  acc = jnp.dot(h_ref[...], ws_ref[...], preferred_element_type=jnp.float32)
  acc = acc + jnp.dot(agg, wn_ref[...], preferred_element_type=jnp.float32)
  o_ref[...] = jnp.maximum(acc + b_ref[...], 0.0)


def _tc_combine(h, partials, w_self, w_neigh, b):
  blk = 2000
  grid = (N_NODES // blk,)
  return pl.pallas_call(
      _tc_combine_body,
      grid=grid,
      in_specs=[
          pl.BlockSpec((blk, D), lambda i: (i, 0)),
          pl.BlockSpec((NC, blk, D), lambda i: (0, i, 0)),
          pl.BlockSpec((D, D), lambda i: (0, 0)),
          pl.BlockSpec((D, D), lambda i: (0, 0)),
          pl.BlockSpec((1, D), lambda i: (0, 0)),
      ],
      out_specs=pl.BlockSpec((blk, D), lambda i: (i, 0)),
      out_shape=jax.ShapeDtypeStruct((N_NODES, D), jnp.float32),
  )(h, partials, w_self, w_neigh, b.reshape(1, D))


def kernel(h, edge_index, W_self, W_neigh, b, index):
  del index
  edges = edge_index.astype(jnp.int32)
  zeros_chunk = jnp.zeros((ROWS_PER_TILE, D), jnp.float32)
  partials = _sc_segment_sum(edges, h, zeros_chunk)
  return _tc_combine(h, partials, W_self, W_neigh, b)


# speedup vs baseline: 2.4422x; 1.0089x over previous
import functools

import jax
import jax.numpy as jnp
from jax import lax
from jax.experimental import pallas as pl
from jax.experimental.pallas import tpu as pltpu
from jax.experimental.pallas import tpu_sc as plsc

N_NODES = 10000
N_EDGES = 320000
D = 128

NC = 2
NS = 16
B = 128
CHUNKS0 = 79
CHUNKS1 = 78
E0 = NS * CHUNKS0 * B
CHUNKS1_LAST = (N_EDGES - E0) // B - (NS - 1) * CHUNKS1
ROWS_PER_TILE = 632
ACC_ROWS = NS * ROWS_PER_TILE
OUT_ROWS_PER_TILE = 624
OUT_ROWS_LAST = N_NODES - 15 * OUT_ROWS_PER_TILE


def _sc_segment_sum(edges, h, zeros_chunk):
  mesh = plsc.VectorSubcoreMesh(core_axis_name="c", subcore_axis_name="s")

  @functools.partial(
      pl.kernel,
      mesh=mesh,
      out_type=jax.ShapeDtypeStruct((NC, N_NODES, D), jnp.float32),
      scratch_types=[
          pltpu.VMEM((CHUNKS0 * B,), jnp.int32),
          pltpu.VMEM((CHUNKS0 * B,), jnp.int32),
          pltpu.VMEM((B, D), jnp.float32),
          pltpu.VMEM_SHARED((ACC_ROWS, D), jnp.float32),
          pltpu.SemaphoreType.DMA,
          pltpu.SemaphoreType.DMA,
          pltpu.SemaphoreType.DMA,
      ],
  )
  def seg_sum(edges_hbm, h_hbm, z_hbm, out_hbm,
              src_v, dst_v, rows_v, acc_sh, sem, sem_s, sem_d):
    c = lax.axis_index("c")
    s = lax.axis_index("s")

    start = jnp.where(c == 0, s * (CHUNKS0 * B),
                      E0 + s * (CHUNKS1 * B))
    startc = jnp.minimum(start, N_EDGES - CHUNKS0 * B)
    off = (start - startc) // B
    cp_s = pltpu.async_copy(edges_hbm.at[0, pl.ds(startc, CHUNKS0 * B)],
                            src_v, sem_s)
    cp_d = pltpu.async_copy(edges_hbm.at[1, pl.ds(startc, CHUNKS0 * B)],
                            dst_v, sem_d)

    pltpu.sync_copy(z_hbm, acc_sh.at[pl.ds(s * ROWS_PER_TILE, ROWS_PER_TILE)])
    cp_s.wait()
    cp_d.wait()
    plsc.subcore_barrier()

    def body(jj, carry):
      j = jj + off
      pltpu.async_copy(h_hbm.at[src_v.at[pl.ds(j * B, B)]], rows_v, sem).wait()
      pltpu.sync_copy(rows_v, acc_sh.at[dst_v.at[pl.ds(j * B, B)]], add=True)
      return carry

    nchunks = jnp.where(c == 0, CHUNKS0,
                        jnp.where(s == NS - 1, CHUNKS1_LAST, CHUNKS1))
    lax.fori_loop(0, nchunks, body, 0)
    plsc.subcore_barrier()

    base = s * OUT_ROWS_PER_TILE

    @pl.when(s < NS - 1)
    def _():
      pltpu.sync_copy(acc_sh.at[pl.ds(base, OUT_ROWS_PER_TILE)],
                      out_hbm.at[c, pl.ds(base, OUT_ROWS_PER_TILE)])

    @pl.when(s == NS - 1)
    def _():
      last = (NS - 1) * OUT_ROWS_PER_TILE
      pltpu.sync_copy(acc_sh.at[pl.ds(last, OUT_ROWS_LAST)],
                      out_hbm.at[c, pl.ds(last, OUT_ROWS_LAST)])

  return seg_sum(edges, h, zeros_chunk)


def _tc_combine_body(h_ref, p_ref, ws_ref, wn_ref, b_ref, o_ref):
  agg = p_ref[0] + p_ref[1]
  acc = jnp.dot(h_ref[...], ws_ref[...], preferred_element_type=jnp.float32)
  acc = acc + jnp.dot(agg, wn_ref[...], preferred_element_type=jnp.float32)
  o_ref[...] = jnp.maximum(acc + b_ref[...], 0.0)


def _tc_combine(h, partials, w_self, w_neigh, b):
  blk = 2000
  grid = (N_NODES // blk,)
  return pl.pallas_call(
      _tc_combine_body,
      grid=grid,
      in_specs=[
          pl.BlockSpec((blk, D), lambda i: (i, 0)),
          pl.BlockSpec((NC, blk, D), lambda i: (0, i, 0)),
          pl.BlockSpec((D, D), lambda i: (0, 0)),
          pl.BlockSpec((D, D), lambda i: (0, 0)),
          pl.BlockSpec((1, D), lambda i: (0, 0)),
      ],
      out_specs=pl.BlockSpec((blk, D), lambda i: (i, 0)),
      out_shape=jax.ShapeDtypeStruct((N_NODES, D), jnp.float32),
  )(h, partials, w_self, w_neigh, b.reshape(1, D))


def kernel(h, edge_index, W_self, W_neigh, b, index):
  del index
  edges = edge_index.astype(jnp.int32)
  zeros_chunk = jnp.zeros((ROWS_PER_TILE, D), jnp.float32)
  partials = _sc_segment_sum(edges, h, zeros_chunk)
  return _tc_combine(h, partials, W_self, W_neigh, b)
